# Initial kernel scaffold; baseline (speedup 1.0000x reference)
import jax
import jax.numpy as jnp
from jax.experimental import pallas as pl


def kernel(user_emb, item_emb, edge_index, edge_weight):
    raise NotImplementedError("write your pallas kernel here")


# speedup vs baseline: 3.0921x; 3.0921x over previous
import functools
import math

import jax
import jax.numpy as jnp
from jax import lax
from jax.experimental import pallas as pl
from jax.experimental.pallas import tpu as pltpu
from jax.experimental.pallas import tpu_sc as plsc

NUM_USERS = 12500
NUM_ITEMS = 37500
N = NUM_USERS + NUM_ITEMS
E = 800000
D = 64
NL = 3
NC = 2
NS = 16
LANES = 16

HALF = N // NC
SUB = 128
EB = 256
NSUB = EB // SUB
NB = math.ceil(E / (NS * EB))
EPW = EB * NB
EPAD = NS * EPW

TRASH = NS * LANES
ACC_ROWS = HALF + TRASH
RCH = 1568

_mesh = plsc.VectorSubcoreMesh(
    core_axis_name="c", subcore_axis_name="s", num_cores=NC, num_subcores=NS
)


def _layer_body(x_hbm, src_hbm, dst_hbm, w_hbm, out_hbm,
                srcv, dstv, idxv, wv, rowsv, acc, sem):
    c = lax.axis_index("c")
    s = lax.axis_index("s")

    @pl.loop(0, EB)
    def _zero_rows(r):
        for k in range(D // LANES):
            rowsv[r, pl.ds(k * LANES, LANES)] = jnp.zeros((LANES,), jnp.float32)

    zstart = jnp.minimum(s * RCH, HALF - RCH)
    n_full, rem = RCH // EB, RCH % EB
    for z in range(n_full):
        pltpu.sync_copy(rowsv, acc.at[pl.ds(zstart + z * EB, EB)])
    if rem:
        pltpu.sync_copy(rowsv.at[pl.ds(0, rem)],
                        acc.at[pl.ds(zstart + n_full * EB, rem)])
    plsc.subcore_barrier()

    iota16 = lax.iota(jnp.int32, LANES)
    trash_base = HALF + s * LANES

    @pl.loop(0, NB)
    def _block(b):
        off = s * EPW + b * EB
        for j in range(NSUB):
            pltpu.sync_copy(src_hbm.at[pl.ds(off + j * SUB, SUB)], srcv.at[j])
            pltpu.sync_copy(dst_hbm.at[pl.ds(off + j * SUB, SUB)], dstv.at[j])
            pltpu.sync_copy(w_hbm.at[pl.ds(off + j * SUB, SUB)], wv.at[j])

        cps = [
            pltpu.async_copy(
                x_hbm.at[srcv.at[j]],
                rowsv.at[pl.ds(j * SUB, SUB)], sem)
            for j in range(NSUB)
        ]
        for cp in cps:
            cp.wait()

        for j in range(NSUB):
            @pl.loop(0, SUB // LANES)
            def _dst16(t, j=j):
                d16 = dstv[j, pl.ds(t * LANES, LANES)] - c * HALF
                m = (d16 >= 0) & (d16 < HALF)
                idxv[j, pl.ds(t * LANES, LANES)] = jnp.where(
                    m, d16, trash_base + iota16)

        for j in range(NSUB):
            @pl.loop(0, SUB // LANES)
            def _scale(g, j=j):
                w16 = wv[j, pl.ds(g * LANES, LANES)]
                e0 = j * SUB + g * LANES
                for l in range(LANES):
                    wb = lax.broadcast(w16[l], (LANES,))
                    for k in range(D // LANES):
                        sl = pl.ds(k * LANES, LANES)
                        rowsv[e0 + l, sl] = rowsv[e0 + l, sl] * wb

        for j in range(NSUB):
            pltpu.sync_copy(rowsv.at[pl.ds(j * SUB, SUB)],
                            acc.at[idxv.at[j]], add=True)

    plsc.subcore_barrier()

    rstart = jnp.minimum(s * RCH, HALF - RCH)
    pltpu.sync_copy(acc.at[pl.ds(rstart, RCH)],
                    out_hbm.at[pl.ds(c * HALF + rstart, RCH)])


def _propagate_layer(x, src2d, dst2d, w2d):
    kern = pl.kernel(
        _layer_body,
        out_type=jax.ShapeDtypeStruct((N, D), jnp.float32),
        mesh=_mesh,
        scratch_types=[
            pltpu.VMEM((NSUB, SUB), jnp.int32),
            pltpu.VMEM((NSUB, SUB), jnp.int32),
            pltpu.VMEM((NSUB, SUB), jnp.int32),
            pltpu.VMEM((NSUB, SUB), jnp.float32),
            pltpu.VMEM((EB, D), jnp.float32),
            pltpu.VMEM_SHARED((ACC_ROWS, D), jnp.float32),
            pltpu.SemaphoreType.DMA,
        ],
        compiler_params=pltpu.CompilerParams(use_tc_tiling_on_sc=False),
    )
    return kern(x, src2d, dst2d, w2d)


def _mean_body(a_ref, b_ref, c_ref, d_ref, o_ref):
    o_ref[...] = (a_ref[...] + b_ref[...] + c_ref[...] + d_ref[...]) * 0.25


def _mean4(x0, x1, x2, x3):
    blk = 2000
    spec = pl.BlockSpec((blk, D), lambda i: (i, 0))
    return pl.pallas_call(
        _mean_body,
        grid=(N // blk,),
        in_specs=[spec] * 4,
        out_specs=spec,
        out_shape=jax.ShapeDtypeStruct((N, D), jnp.float32),
    )(x0, x1, x2, x3)


def kernel(user_emb, item_emb, edge_index, edge_weight):
    x0 = jnp.concatenate([user_emb, item_emb], axis=0)
    src = edge_index[0]
    dst = edge_index[1]

    pad = EPAD - E
    ar = jnp.arange(pad, dtype=jnp.int32)
    src_p = jnp.concatenate([src, (ar * 61) % N])
    dst_p = jnp.concatenate([dst, (ar * 97) % N])
    w_p = jnp.concatenate([edge_weight, jnp.zeros((pad,), jnp.float32)])

    x1 = _propagate_layer(x0, src_p, dst_p, w_p)
    x2 = _propagate_layer(x1, src_p, dst_p, w_p)
    x3 = _propagate_layer(x2, src_p, dst_p, w_p)
    final = _mean4(x0, x1, x2, x3)
    return final[:NUM_USERS], final[NUM_USERS:]


# speedup vs baseline: 8.2904x; 2.6812x over previous
import math

import jax
import jax.numpy as jnp
from jax import lax
from jax.experimental import pallas as pl
from jax.experimental.pallas import tpu as pltpu
from jax.experimental.pallas import tpu_sc as plsc

NUM_USERS = 12500
NUM_ITEMS = 37500
N = NUM_USERS + NUM_ITEMS
E = 800000
D = 64
NC = 2
NS = 16
LANES = 16

HALF = N // NC
CH = 128
NSLOT = 3
NU = 131
NCHUNK = NSLOT * NU
EPW = CH * NCHUNK
EPAD = NS * EPW

TRASH = NS * LANES
ACC_ROWS = HALF + TRASH
RCH = 1568

_mesh = plsc.VectorSubcoreMesh(
    core_axis_name="c", subcore_axis_name="s", num_cores=NC, num_subcores=NS
)


def _layer_body(x_hbm, src_hbm, dst_hbm, w_hbm, out_hbm,
                srcv, dstv, idxv, wv, rowsv, acc,
                sem_g, sem_s, sem_i):
    c = lax.axis_index("c")
    s = lax.axis_index("s")
    ebase = s * EPW

    @pl.loop(0, CH)
    def _zero_rows(r):
        for k in range(D // LANES):
            rowsv[0, r, pl.ds(k * LANES, LANES)] = jnp.zeros(
                (LANES,), jnp.float32)

    zstart = jnp.minimum(s * RCH, HALF - RCH)
    n_full, rem = RCH // CH, RCH % CH
    for z in range(n_full):
        pltpu.sync_copy(rowsv.at[0], acc.at[pl.ds(zstart + z * CH, CH)])
    if rem:
        pltpu.sync_copy(rowsv.at[0, pl.ds(0, rem)],
                        acc.at[pl.ds(zstart + n_full * CH, rem)])
    plsc.subcore_barrier()

    iota16 = lax.iota(jnp.int32, LANES)
    trash16 = HALF + s * LANES + iota16

    def fire_idx(i, p):
        off = ebase + i * CH
        pltpu.async_copy(src_hbm.at[pl.ds(off, CH)], srcv.at[p], sem_i[p])
        pltpu.async_copy(dst_hbm.at[pl.ds(off, CH)], dstv.at[p], sem_i[p])
        pltpu.async_copy(w_hbm.at[pl.ds(off, CH)], wv.at[p], sem_i[p])

    def wait_idx(p):
        pltpu.make_async_copy(src_hbm.at[pl.ds(0, CH)], srcv.at[p],
                              sem_i[p]).wait()
        pltpu.make_async_copy(dst_hbm.at[pl.ds(0, CH)], dstv.at[p],
                              sem_i[p]).wait()
        pltpu.make_async_copy(w_hbm.at[pl.ds(0, CH)], wv.at[p],
                              sem_i[p]).wait()

    def fire_gather(p):
        pltpu.async_copy(x_hbm.at[srcv.at[p]], rowsv.at[p], sem_g[p])

    def wait_gather(p):
        pltpu.make_async_copy(x_hbm.at[srcv.at[p]], rowsv.at[p],
                              sem_g[p]).wait()

    def fire_scatter(p):
        pltpu.async_copy(rowsv.at[p], acc.at[idxv.at[p]], sem_s[p],
                         add=True)

    def wait_scatter(p):
        pltpu.make_async_copy(rowsv.at[p], acc.at[idxv.at[p]],
                              sem_s[p]).wait()

    def compute(p):
        @pl.loop(0, CH // LANES)
        def _dst16(t):
            d16 = dstv[p, pl.ds(t * LANES, LANES)] - c * HALF
            m = (d16 >= 0) & (d16 < HALF)
            idxv[p, pl.ds(t * LANES, LANES)] = jnp.where(m, d16, trash16)

        @pl.loop(0, CH // LANES)
        def _scale(g):
            w16 = wv[p, pl.ds(g * LANES, LANES)]
            e0 = g * LANES
            for l in range(LANES):
                wb = lax.broadcast(w16[l], (LANES,))
                for k in range(D // LANES):
                    sl = pl.ds(k * LANES, LANES)
                    rowsv[p, e0 + l, sl] = rowsv[p, e0 + l, sl] * wb

    for p in range(NSLOT):
        fire_idx(p, p)
    wait_idx(0)
    fire_gather(0)

    def step(i, d, u, first):
        p = d
        q = (d + 1) % NSLOT
        wait_gather(p)
        if not (first and d < 2):
            wait_scatter(q)
        if first:
            wait_idx(q)
            fire_gather(q)
        elif d < 2:
            wait_idx(q)
            fire_gather(q)
        else:
            @pl.when(u < NU - 1)
            def _next_gather():
                wait_idx(q)
                fire_gather(q)
        compute(p)
        fire_scatter(p)
        if first:
            fire_idx(i + NSLOT, p)
        else:
            @pl.when(u < NU - 1)
            def _prefetch():
                fire_idx(i + NSLOT, p)

    for d in range(NSLOT):
        step(d, d, 0, True)

    @pl.loop(1, NU)
    def _ring(u):
        for d in range(NSLOT):
            step(u * NSLOT + d, d, u, False)

    wait_scatter((NCHUNK - 2) % NSLOT)
    wait_scatter((NCHUNK - 1) % NSLOT)
    plsc.subcore_barrier()

    rstart = jnp.minimum(s * RCH, HALF - RCH)
    pltpu.sync_copy(acc.at[pl.ds(rstart, RCH)],
                    out_hbm.at[pl.ds(c * HALF + rstart, RCH)])


def _propagate_layer(x, src1d, dst1d, w1d):
    kern = pl.kernel(
        _layer_body,
        out_type=jax.ShapeDtypeStruct((N, D), jnp.float32),
        mesh=_mesh,
        scratch_types=[
            pltpu.VMEM((NSLOT, CH), jnp.int32),
            pltpu.VMEM((NSLOT, CH), jnp.int32),
            pltpu.VMEM((NSLOT, CH), jnp.int32),
            pltpu.VMEM((NSLOT, CH), jnp.float32),
            pltpu.VMEM((NSLOT, CH, D), jnp.float32),
            pltpu.VMEM_SHARED((ACC_ROWS, D), jnp.float32),
            [pltpu.SemaphoreType.DMA] * NSLOT,
            [pltpu.SemaphoreType.DMA] * NSLOT,
            [pltpu.SemaphoreType.DMA] * NSLOT,
        ],
        compiler_params=pltpu.CompilerParams(use_tc_tiling_on_sc=False),
    )
    return kern(x, src1d, dst1d, w1d)


def _mean_body(a_ref, b_ref, c_ref, d_ref, o_ref):
    o_ref[...] = (a_ref[...] + b_ref[...] + c_ref[...] + d_ref[...]) * 0.25


def _mean4(x0, x1, x2, x3):
    blk = 2000
    spec = pl.BlockSpec((blk, D), lambda i: (i, 0))
    return pl.pallas_call(
        _mean_body,
        grid=(N // blk,),
        in_specs=[spec] * 4,
        out_specs=spec,
        out_shape=jax.ShapeDtypeStruct((N, D), jnp.float32),
    )(x0, x1, x2, x3)


def kernel(user_emb, item_emb, edge_index, edge_weight):
    x0 = jnp.concatenate([user_emb, item_emb], axis=0)
    src = edge_index[0]
    dst = edge_index[1]

    pad = EPAD - E
    ar = jnp.arange(pad, dtype=jnp.int32)
    src_p = jnp.concatenate([src, (ar * 61) % N])
    dst_p = jnp.concatenate([dst, (ar * 97) % N])
    w_p = jnp.concatenate([edge_weight, jnp.zeros((pad,), jnp.float32)])

    x1 = _propagate_layer(x0, src_p, dst_p, w_p)
    x2 = _propagate_layer(x1, src_p, dst_p, w_p)
    x3 = _propagate_layer(x2, src_p, dst_p, w_p)
    final = _mean4(x0, x1, x2, x3)
    return final[:NUM_USERS], final[NUM_USERS:]
